# baseline (speedup 1.0000x reference)
import functools

import jax
import jax.numpy as jnp
from jax import lax
from jax.experimental import pallas as pl
from jax.experimental.pallas import tpu as pltpu
from jax.experimental.pallas import tpu_sc as plsc

D_MODEL = 64
SCALE = 8.0
NC, NS = 2, 16
NW = NC * NS
CB = 128
L = 16
NBUF = 4


@functools.cache
def _make_transpose(v):
    d = D_MODEL
    CBA = 2 * CB
    nfull = v // CBA
    tail = v - nfull * CBA
    mesh = plsc.VectorSubcoreMesh(core_axis_name="c", subcore_axis_name="s")

    @functools.partial(
        pl.kernel,
        out_type=jax.ShapeDtypeStruct((v * d // CB, CB), jnp.float32),
        mesh=mesh,
        scratch_types=[
            [pltpu.VMEM((d, CBA), jnp.float32)] * 3,
            [pltpu.VMEM((d * CBA // 128, 128), jnp.float32)] * 3,
            [pltpu.SemaphoreType.DMA] * 3,
            [pltpu.SemaphoreType.DMA] * 3,
        ],
        compiler_params=pltpu.CompilerParams(
            use_tc_tiling_on_sc=True, needs_layout_passes=False
        ),
    )
    def trans_kernel(lutt_hbm, out_hbm, inb, outb, isem, osem):
        w = lax.axis_index("s") * NC + lax.axis_index("c")
        iota = lax.iota(jnp.int32, L)
        nstep = (nfull - w + NW - 1) // NW

        def fire_in(step, b):
            t0 = pl.multiple_of(step * CBA, CB)
            for fb in range(d // 8):
                pltpu.async_copy(
                    lutt_hbm.at[pl.ds(fb * 8, 8), pl.ds(t0, CBA)],
                    inb[b].at[pl.ds(fb * 8, 8)],
                    isem[b],
                )

        def fire_in_tail(b):
            t0 = pl.multiple_of(w * 0 + nfull * CBA, CB)
            for fb in range(d // 8):
                pltpu.async_copy(
                    lutt_hbm.at[pl.ds(fb * 8, 8), pl.ds(t0, CB)],
                    inb[b].at[pl.ds(fb * 8, 8), pl.ds(0, CB)],
                    isem[b],
                )

        def wait_in(b, nt):
            pltpu.make_async_copy(
                out_hbm.at[pl.ds(0, d // 2)],
                inb[b].at[:, pl.ds(0, nt)],
                isem[b],
            ).wait()

        def transpose(b, ntg):
            @plsc.parallel_loop(0, ntg * L, step=1, unroll=8)
            def _(i):
                tg = lax.shift_right_logical(i, 4)
                k = lax.bitwise_and(i, 15)
                t_vec = tg * L + iota
                f_base = lax.bitwise_and(iota + k, 15)
                row = lax.shift_right_logical(t_vec, 1)
                half = lax.bitwise_and(t_vec, 1) * d
                for c in range(d // L):
                    f_vec = c * L + f_base
                    vals = plsc.load_gather(inb[b], [f_vec, t_vec])
                    plsc.store_scatter(outb[b], [row, half + f_vec], vals)

        def fire_out(step, b, nt):
            pltpu.async_copy(
                outb[b].at[pl.ds(0, nt * d // 128)],
                out_hbm.at[pl.ds(step * (CBA * d // 128), nt * d // 128)],
                osem[b],
            )

        def wait_out(b, nt):
            pltpu.make_async_copy(
                out_hbm.at[pl.ds(0, nt * d // 128)],
                outb[b].at[pl.ds(0, nt * d // 128)],
                osem[b],
            ).wait()

        @pl.when(nstep > 0)
        def _():
            fire_in(w, 0)

        def group_body(g3, carry):
            for bb in range(3):
                gi = g3 * 3 + bb

                @pl.when(gi < nstep)
                def _():
                    step = w + gi * NW

                    @pl.when(gi > 2)
                    def _():
                        wait_out(bb, CBA)

                    wait_in(bb, CBA)
                    transpose(bb, CBA // L)
                    fire_out(step, bb, CBA)

                    @pl.when(gi + 1 < nstep)
                    def _():
                        fire_in(step + NW, (bb + 1) % 3)

            return carry

        lax.fori_loop(0, (nstep + 2) // 3, group_body, 0)

        for bb in range(3):
            @pl.when(nstep > bb)
            def _():
                wait_out(bb, CBA)

        if tail:
            @pl.when(w == 0)
            def _():
                fire_in_tail(0)
                wait_in(0, CB)
                transpose(0, CB // L)
                fire_out(nfull, 0, tail)
                wait_out(0, tail)

    return trans_kernel


@functools.cache
def _make_kernel(n, s, v):
    assert n == NW * CB, (n, NW * CB)
    nfb = D_MODEL // 8
    mesh = plsc.VectorSubcoreMesh(core_axis_name="c", subcore_axis_name="s")

    @functools.partial(
        pl.kernel,
        out_type=jax.ShapeDtypeStruct((s, nfb, NW, 8, CB), jnp.float32),
        mesh=mesh,
        scratch_types=[
            pltpu.VMEM((s, CB), jnp.int32),
            [pltpu.VMEM((CB, D_MODEL), jnp.float32)] * NBUF,
            [pltpu.VMEM((D_MODEL, CB + 1), jnp.float32)] * NBUF,
            [pltpu.SemaphoreType.DMA] * NBUF,
            [pltpu.SemaphoreType.DMA] * NBUF,
        ],
        compiler_params=pltpu.CompilerParams(
            use_tc_tiling_on_sc=False, needs_layout_passes=False
        ),
    )
    def emb_kernel(xt_hbm, lut_hbm, out_hbm, idx_v, rows, tiles, gsem, wsem):
        w = lax.axis_index("s") * NC + lax.axis_index("c")
        pltpu.sync_copy(xt_hbm.at[:, pl.ds(w * CB, CB)], idx_v)

        iota = lax.iota(jnp.int32, L)

        def fire_gather(j, b):
            pltpu.async_copy(lut_hbm.at[idx_v.at[j]], rows[b], gsem[b])

        def wait_gather(b):
            pltpu.make_async_copy(out_hbm.at[0, 0, 0], rows[b], gsem[b]).wait()

        def fire_writes(j, b):
            for rb in range(nfb):
                pltpu.async_copy(
                    tiles[b].at[pl.ds(rb * 8, 8), pl.ds(0, CB)],
                    out_hbm.at[j, rb, w],
                    wsem[b],
                )

        def wait_writes(b):
            pltpu.make_async_copy(
                out_hbm.at[0], tiles[b].at[:, pl.ds(0, CB)], wsem[b]
            ).wait()

        for b in range(NBUF):
            fire_gather(b, b)

        ngrp = s // NBUF

        def group_body(g, carry):
            for b in range(NBUF):
                j = g * NBUF + b

                @pl.when(g > 0)
                def _():
                    wait_writes(b)

                wait_gather(b)

                @plsc.parallel_loop(0, CB, step=1, unroll=4)
                def _(r):
                    for c in range(D_MODEL // L):
                        vals = rows[b][r, pl.ds(c * L, L)] * SCALE
                        plsc.store_scatter(
                            tiles[b],
                            [c * L + iota, jnp.broadcast_to(r, (L,))],
                            vals,
                        )

                fire_writes(j, b)

                @pl.when(g + 1 < ngrp)
                def _():
                    fire_gather(j + NBUF, b)

            return carry

        lax.fori_loop(0, ngrp, group_body, 0)
        for b in range(NBUF):
            wait_writes(b)

    return emb_kernel


def kernel(x, lut):
    n, s = x.shape
    v, d = lut.shape
    xt = x.T.astype(jnp.int32)
    lut_packed = _make_transpose(v)(lut.T)
    out5 = _make_kernel(n, s, v)(xt, lut_packed.reshape(v, d))
    return out5.transpose(2, 4, 0, 1, 3).reshape(n, s, d)


# speedup vs baseline: 1.1126x; 1.1126x over previous
import functools

import jax
import jax.numpy as jnp
from jax import lax
from jax.experimental import pallas as pl
from jax.experimental.pallas import tpu as pltpu
from jax.experimental.pallas import tpu_sc as plsc

D_MODEL = 64
SCALE = 8.0
NC, NS = 2, 16
NW = NC * NS
CB = 128
L = 16
NBUF = 4


@functools.cache
def _make_transpose(v):
    d = D_MODEL
    CBA = 2 * CB
    nfull = v // CBA
    tail = v - nfull * CBA
    mesh = plsc.VectorSubcoreMesh(core_axis_name="c", subcore_axis_name="s")

    @functools.partial(
        pl.kernel,
        out_type=jax.ShapeDtypeStruct((v * d // CB, CB), jnp.float32),
        mesh=mesh,
        scratch_types=[
            [pltpu.VMEM((d, CBA), jnp.float32)] * 3,
            [pltpu.VMEM((d * CBA // 128, 128), jnp.float32)] * 3,
            [pltpu.SemaphoreType.DMA] * 3,
            [pltpu.SemaphoreType.DMA] * 3,
        ],
        compiler_params=pltpu.CompilerParams(
            use_tc_tiling_on_sc=True, needs_layout_passes=False
        ),
    )
    def trans_kernel(lutt_hbm, out_hbm, inb, outb, isem, osem):
        w = lax.axis_index("s") * NC + lax.axis_index("c")
        iota = lax.iota(jnp.int32, L)
        nstep = (nfull - w + NW - 1) // NW

        def fire_in(step, b):
            t0 = pl.multiple_of(step * CBA, CB)
            for fb in range(d // 8):
                pltpu.async_copy(
                    lutt_hbm.at[pl.ds(fb * 8, 8), pl.ds(t0, CBA)],
                    inb[b].at[pl.ds(fb * 8, 8)],
                    isem[b],
                )

        def fire_in_tail(b):
            t0 = pl.multiple_of(w * 0 + nfull * CBA, CB)
            for fb in range(d // 8):
                pltpu.async_copy(
                    lutt_hbm.at[pl.ds(fb * 8, 8), pl.ds(t0, CB)],
                    inb[b].at[pl.ds(fb * 8, 8), pl.ds(0, CB)],
                    isem[b],
                )

        def wait_in(b, nt):
            pltpu.make_async_copy(
                out_hbm.at[pl.ds(0, d // 2)],
                inb[b].at[:, pl.ds(0, nt)],
                isem[b],
            ).wait()

        def transpose(b, ntg):
            @plsc.parallel_loop(0, ntg * L, step=1, unroll=4)
            def _(i):
                tg = lax.shift_right_logical(i, 4)
                k = lax.bitwise_and(i, 15)
                t_vec = tg * L + iota
                f_base = lax.bitwise_and(iota + k, 15)
                row = lax.shift_right_logical(t_vec, 1)
                half = lax.bitwise_and(t_vec, 1) * d
                for c in range(d // L):
                    f_vec = c * L + f_base
                    vals = plsc.load_gather(inb[b], [f_vec, t_vec])
                    plsc.store_scatter(outb[b], [row, half + f_vec], vals)

        def fire_out(step, b, nt):
            pltpu.async_copy(
                outb[b].at[pl.ds(0, nt * d // 128)],
                out_hbm.at[pl.ds(step * (CBA * d // 128), nt * d // 128)],
                osem[b],
            )

        def wait_out(b, nt):
            pltpu.make_async_copy(
                out_hbm.at[pl.ds(0, nt * d // 128)],
                outb[b].at[pl.ds(0, nt * d // 128)],
                osem[b],
            ).wait()

        @pl.when(nstep > 0)
        def _():
            fire_in(w, 0)

        def group_body(g3, carry):
            for bb in range(3):
                gi = g3 * 3 + bb

                @pl.when(gi < nstep)
                def _():
                    step = w + gi * NW

                    @pl.when(gi > 2)
                    def _():
                        wait_out(bb, CBA)

                    wait_in(bb, CBA)
                    transpose(bb, CBA // L)
                    fire_out(step, bb, CBA)

                    @pl.when(gi + 1 < nstep)
                    def _():
                        fire_in(step + NW, (bb + 1) % 3)

            return carry

        lax.fori_loop(0, (nstep + 2) // 3, group_body, 0)

        for bb in range(3):
            @pl.when(nstep > bb)
            def _():
                wait_out(bb, CBA)

        if tail:
            @pl.when(w == 0)
            def _():
                fire_in_tail(0)
                wait_in(0, CB)
                transpose(0, CB // L)
                fire_out(nfull, 0, tail)
                wait_out(0, tail)

    return trans_kernel


@functools.cache
def _make_kernel(n, s, v):
    assert n == NW * CB, (n, NW * CB)
    nfb = D_MODEL // 8
    mesh = plsc.VectorSubcoreMesh(core_axis_name="c", subcore_axis_name="s")

    @functools.partial(
        pl.kernel,
        out_type=jax.ShapeDtypeStruct((s, nfb, NW, 8, CB), jnp.float32),
        mesh=mesh,
        scratch_types=[
            pltpu.VMEM((s, CB), jnp.int32),
            [pltpu.VMEM((CB, D_MODEL), jnp.float32)] * NBUF,
            [pltpu.VMEM((D_MODEL, CB + 1), jnp.float32)] * NBUF,
            [pltpu.SemaphoreType.DMA] * NBUF,
            [pltpu.SemaphoreType.DMA] * NBUF,
        ],
        compiler_params=pltpu.CompilerParams(
            use_tc_tiling_on_sc=False, needs_layout_passes=False
        ),
    )
    def emb_kernel(xt_hbm, lut_hbm, out_hbm, idx_v, rows, tiles, gsem, wsem):
        w = lax.axis_index("s") * NC + lax.axis_index("c")
        pltpu.sync_copy(xt_hbm.at[:, pl.ds(w * CB, CB)], idx_v)

        iota = lax.iota(jnp.int32, L)

        def fire_gather(j, b):
            pltpu.async_copy(lut_hbm.at[idx_v.at[j]], rows[b], gsem[b])

        def wait_gather(b):
            pltpu.make_async_copy(out_hbm.at[0, 0, 0], rows[b], gsem[b]).wait()

        def fire_writes(j, b):
            for rb in range(nfb):
                pltpu.async_copy(
                    tiles[b].at[pl.ds(rb * 8, 8), pl.ds(0, CB)],
                    out_hbm.at[j, rb, w],
                    wsem[b],
                )

        def wait_writes(b):
            pltpu.make_async_copy(
                out_hbm.at[0], tiles[b].at[:, pl.ds(0, CB)], wsem[b]
            ).wait()

        for b in range(NBUF):
            fire_gather(b, b)

        ngrp = s // NBUF

        def group_body(g, carry):
            for b in range(NBUF):
                j = g * NBUF + b

                @pl.when(g > 0)
                def _():
                    wait_writes(b)

                wait_gather(b)

                @plsc.parallel_loop(0, CB, step=1, unroll=4)
                def _(r):
                    for c in range(D_MODEL // L):
                        vals = rows[b][r, pl.ds(c * L, L)] * SCALE
                        plsc.store_scatter(
                            tiles[b],
                            [c * L + iota, jnp.broadcast_to(r, (L,))],
                            vals,
                        )

                fire_writes(j, b)

                @pl.when(g + 1 < ngrp)
                def _():
                    fire_gather(j + NBUF, b)

            return carry

        lax.fori_loop(0, ngrp, group_body, 0)
        for b in range(NBUF):
            wait_writes(b)

    return emb_kernel


def kernel(x, lut):
    n, s = x.shape
    v, d = lut.shape
    xt = x.T.astype(jnp.int32)
    lut_packed = _make_transpose(v)(lut.T)
    out5 = _make_kernel(n, s, v)(xt, lut_packed.reshape(v, d))
    return out5.transpose(2, 4, 0, 1, 3).reshape(n, s, d)
